# Initial kernel scaffold; baseline (speedup 1.0000x reference)
import jax
import jax.numpy as jnp
from jax.experimental import pallas as pl


def kernel(z, codebook):
    raise NotImplementedError("write your pallas kernel here")


# speedup vs baseline: 2.3059x; 2.3059x over previous
import jax
import jax.numpy as jnp
from jax.experimental import pallas as pl
from jax.experimental.pallas import tpu as pltpu

_K = 1024
_D = 64
_BLK = 512


def _vq_block(z_ref, zsum_ref, cb_ref, csum_ref,
              dist_ref, enc_ref, idx_ref, zq_ref, loss_ref, plex_ref,
              counts_ref, err_ref):
    i = pl.program_id(0)
    g = pl.num_programs(0)
    z = z_ref[...]
    cb = cb_ref[...]
    m = jax.lax.dot_general(z, cb, (((1,), (1,)), ((), ())),
                            preferred_element_type=jnp.float32)
    d = (zsum_ref[...] + csum_ref[...]) - 2.0 * m
    dist_ref[...] = d

    mn = jnp.min(d, axis=1, keepdims=True)
    iota = jax.lax.broadcasted_iota(jnp.int32, (_BLK, _K), 1)
    idx = jnp.min(jnp.where(d == mn, iota, _K), axis=1)
    idx_ref[0, 0, :] = idx

    enc = (iota == idx[:, None]).astype(jnp.float32)
    enc_ref[...] = enc
    zq = jax.lax.dot_general(enc, cb, (((1,), (0,)), ((), ())),
                             preferred_element_type=jnp.float32)
    zq_ref[...] = zq

    e = zq - z
    blk_err = jnp.sum(e * e)
    blk_counts = jnp.sum(enc, axis=0, keepdims=True)

    @pl.when(i == 0)
    def _init():
        err_ref[0, 0] = 0.0
        counts_ref[...] = jnp.zeros_like(counts_ref)

    err_ref[0, 0] += blk_err
    counts_ref[...] += blk_counts

    @pl.when(i == g - 1)
    def _final():
        n_total = g * _BLK
        p = counts_ref[...] * (1.0 / n_total)
        plex_ref[0, 0] = jnp.exp(-jnp.sum(p * jnp.log(p + 1e-10)))
        mse = err_ref[0, 0] / (n_total * _D)
        loss_ref[0, 0] = 0.25 * mse + mse


def kernel(z, codebook):
    b, d, h, w = z.shape
    n = b * h * w
    z_flat = jnp.transpose(z, (0, 2, 3, 1)).reshape(n, d)
    zsum = jnp.sum(z_flat ** 2, axis=1, keepdims=True)
    csum = jnp.sum(codebook ** 2, axis=1)[None, :]
    grid = n // _BLK

    out_shapes = (
        jax.ShapeDtypeStruct((n, _K), jnp.float32),
        jax.ShapeDtypeStruct((n, _K), jnp.float32),
        jax.ShapeDtypeStruct((grid, 1, _BLK), jnp.int32),
        jax.ShapeDtypeStruct((n, _D), jnp.float32),
        jax.ShapeDtypeStruct((1, 1), jnp.float32),
        jax.ShapeDtypeStruct((1, 1), jnp.float32),
    )
    dist, enc, idx3, zqf, loss, plex = pl.pallas_call(
        _vq_block,
        grid=(grid,),
        in_specs=[
            pl.BlockSpec((_BLK, d), lambda i: (i, 0)),
            pl.BlockSpec((_BLK, 1), lambda i: (i, 0)),
            pl.BlockSpec((_K, d), lambda i: (0, 0)),
            pl.BlockSpec((1, _K), lambda i: (0, 0)),
        ],
        out_specs=(
            pl.BlockSpec((_BLK, _K), lambda i: (i, 0)),
            pl.BlockSpec((_BLK, _K), lambda i: (i, 0)),
            pl.BlockSpec((1, 1, _BLK), lambda i: (i, 0, 0)),
            pl.BlockSpec((_BLK, d), lambda i: (i, 0)),
            pl.BlockSpec((1, 1), lambda i: (0, 0), memory_space=pltpu.SMEM),
            pl.BlockSpec((1, 1), lambda i: (0, 0), memory_space=pltpu.SMEM),
        ),
        out_shape=out_shapes,
        scratch_shapes=[
            pltpu.VMEM((1, _K), jnp.float32),
            pltpu.SMEM((1, 1), jnp.float32),
        ],
        compiler_params=pltpu.CompilerParams(
            dimension_semantics=("arbitrary",),
        ),
    )(z_flat, zsum, codebook, csum)

    encoding_indices = idx3.reshape(n)
    z_quantized = jnp.transpose(zqf.reshape(b, h, w, d), (0, 3, 1, 2))
    return (z_quantized, loss[0, 0], plex[0, 0], enc, encoding_indices, dist)
